# Initial kernel scaffold; baseline (speedup 1.0000x reference)
import jax
import jax.numpy as jnp
from jax.experimental import pallas as pl


def kernel(memory_features, memory_importance, features, importance, write_idx, sample_idx):
    raise NotImplementedError("write your pallas kernel here")


# speedup vs baseline: 1.9946x; 1.9946x over previous
import functools

import jax
import jax.numpy as jnp
from jax import lax
from jax.experimental import pallas as pl
from jax.experimental.pallas import tpu as pltpu
from jax.experimental.pallas import tpu_sc as plsc

_NC = 2
_NS = 16
_L = 16
_CHUNK = 128
_FIX_ROUNDS = 3


def _iota16():
    return lax.broadcasted_iota(jnp.int32, (_L,), 0)


def _splat(x):
    return jnp.full((_L,), x, jnp.int32)


def _make_sc_call(M, D, B, S):
    assert D % _L == 0
    assert B % (_NS * _CHUNK) == 0
    assert S % (_NC * _NS * _CHUNK) == 0
    wpt = B // _NS
    wk = wpt // _CHUNK
    spw = S // (_NC * _NS)
    sk = spw // _CHUNK
    span = ((M + _NS * 1024 - 1) // (_NS * 1024)) * 1024
    dummy = _NS * span
    table_n = dummy + _L
    fills = 1024

    mesh = plsc.VectorSubcoreMesh(core_axis_name="c", subcore_axis_name="s")

    @functools.partial(
        pl.kernel,
        mesh=mesh,
        out_type=jax.ShapeDtypeStruct((S, D), jnp.float32),
        scratch_types=[
            pltpu.VMEM_SHARED((table_n,), jnp.int32),
            pltpu.VMEM((fills,), jnp.int32),
            pltpu.VMEM((wk, _CHUNK), jnp.int32),
            pltpu.VMEM((wk, _CHUNK), jnp.int32),
            pltpu.VMEM((_CHUNK,), jnp.int32),
            pltpu.VMEM((_CHUNK,), jnp.int32),
            pltpu.VMEM((sk, _CHUNK), jnp.int32),
            pltpu.VMEM((_CHUNK,), jnp.int32),
            pltpu.VMEM((_CHUNK,), jnp.int32),
            pltpu.VMEM((_CHUNK,), jnp.float32),
            pltpu.VMEM((_CHUNK,), jnp.float32),
            pltpu.VMEM((_CHUNK,), jnp.float32),
            pltpu.VMEM((_CHUNK, D), jnp.float32),
            pltpu.VMEM((_CHUNK, D), jnp.float32),
        ],
        compiler_params=pltpu.CompilerParams(
            needs_layout_passes=False, use_tc_tiling_on_sc=False),
    )
    def sc_call(mem_feat, mem_imp, feats, imp, widx_h, sidx_h, out,
                table, fillv, widx_v, jval_v, tvrow, fixrow,
                sidx_v, jwrow, idxbrow, impa, impb, wrow, mema, featb):
        c = lax.axis_index("c")
        s = lax.axis_index("s")
        wid = c * _NS + s
        i16 = _iota16()

        for q in range(fills // _L):
            fillv[pl.ds(q * _L, _L)] = jnp.full((_L,), -1, jnp.int32)
        base = s * span

        def memset_body(q, _):
            pltpu.sync_copy(fillv, table.at[pl.ds(base + q * fills, fills)])
            return _
        lax.fori_loop(0, span // fills, memset_body, None)

        pltpu.sync_copy(widx_h.at[s], widx_v)
        jbase = s * wpt
        for k in range(wk):
            for v in range(_CHUNK // _L):
                jval_v[k, pl.ds(v * _L, _L)] = i16 + (jbase + k * _CHUNK + v * _L)

        plsc.subcore_barrier()

        for k in range(wk):
            pltpu.sync_copy(jval_v.at[k], table.at[widx_v.at[k]])

        for _r in range(_FIX_ROUNDS):
            plsc.subcore_barrier()

            def fix_body(k, _):
                pltpu.sync_copy(table.at[widx_v.at[k]], tvrow)
                for v in range(_CHUNK // _L):
                    cidx = i16 + v * _L
                    jv = plsc.load_gather(jval_v, [_splat(k), cidx])
                    wv = plsc.load_gather(widx_v, [_splat(k), cidx])
                    tv = tvrow[pl.ds(v * _L, _L)]
                    fixrow[pl.ds(v * _L, _L)] = jnp.where(tv < jv, wv, dummy)
                pltpu.sync_copy(jval_v.at[k], table.at[fixrow])
                return _
            lax.fori_loop(0, wk, fix_body, None)

        plsc.subcore_barrier()

        pltpu.sync_copy(sidx_h.at[wid], sidx_v)

        def sample_body(k, _):
            srow = sidx_v.at[k]
            pltpu.sync_copy(table.at[srow], jwrow)
            pltpu.sync_copy(mem_feat.at[srow], mema)
            pltpu.sync_copy(mem_imp.at[srow], impa)
            for v in range(_CHUNK // _L):
                jw = jwrow[pl.ds(v * _L, _L)]
                idxbrow[pl.ds(v * _L, _L)] = jnp.where(jw >= 0, jw, 0)
            pltpu.sync_copy(feats.at[idxbrow], featb)
            pltpu.sync_copy(imp.at[idxbrow], impb)
            for v in range(_CHUNK // _L):
                jw = jwrow[pl.ds(v * _L, _L)]
                wrow[pl.ds(v * _L, _L)] = jnp.where(
                    jw >= 0, impb[pl.ds(v * _L, _L)], impa[pl.ds(v * _L, _L)])

            def row_body(i, _):
                si = _splat(i)
                jw16 = plsc.load_gather(jwrow, [si])
                hit = jw16 >= 0
                w16 = plsc.load_gather(wrow, [si])
                for ccol in range(D // _L):
                    cidx = i16 + ccol * _L
                    a = plsc.load_gather(mema, [si, cidx])
                    b = plsc.load_gather(featb, [si, cidx])
                    r = jnp.where(hit, b, a) * w16
                    plsc.store_scatter(mema, [si, cidx], r)
                return _
            lax.fori_loop(0, _CHUNK, row_body, None)

            pltpu.sync_copy(mema, out.at[pl.ds(wid * spw + k * _CHUNK, _CHUNK)])
            return _
        lax.fori_loop(0, sk, sample_body, None)

    return sc_call


def kernel(memory_features, memory_importance, features, importance,
           write_idx, sample_idx):
    M, D = memory_features.shape
    B = write_idx.shape[0]
    S = sample_idx.shape[0]
    call = _make_sc_call(M, D, B, S)
    widx3 = write_idx.reshape(_NS, B // (_NS * _CHUNK), _CHUNK)
    sidx3 = sample_idx.reshape(_NC * _NS, S // (_NC * _NS * _CHUNK), _CHUNK)
    return call(memory_features, memory_importance, features, importance,
                widx3, sidx3)
